# baseline (speedup 1.0000x reference)
import math

import jax
import jax.numpy as jnp
import numpy as np
from jax.experimental import pallas as pl
from jax.experimental.pallas import tpu as pltpu

_LANES = 128
_TOPK = 10
_MB = 2
_NEG_INF = float("-inf")
_HALF_PI = math.pi / 2.0

_ATAN_COEFS = (1.0, -0.33333322, 0.19999558, -0.14278576, 0.11050771,
               -0.08785043, 0.06685281, -0.04392841, 0.02191294,
               -0.00703067, 0.00105761)


def _atan_nonneg(z):
    inv = z > 1.0
    zz = jnp.where(inv, 1.0 / jnp.maximum(z, 1e-30), z)
    u = zz * zz
    q = jnp.full_like(u, _ATAN_COEFS[-1])
    for c in _ATAN_COEFS[-2::-1]:
        q = q * u + c
    r = zz * q
    return jnp.where(inv, _HALF_PI - r, r)


def _loss_kernel(labels_ref, gt_ref, scores_ref, boxes_ref, anc_ref,
                 out_ref, acc_ref, met_s, iou_s, row_s,
                 px1_s, py1_s, px2_s, py2_s):
    s = pl.program_id(0)
    ns = pl.num_programs(0)
    rows, lanes = anc_ref.shape[1], anc_ref.shape[2]
    num_a = rows * lanes
    nb = ns * _MB
    slab = scores_ref.shape[1] // _MB

    crows = rows
    acc = jnp.zeros((crows, lanes), jnp.float32)
    for i in range(_MB * slab // crows):
        x = scores_ref[0, i * crows:(i + 1) * crows]
        e = jnp.exp2(jnp.abs(x) * -1.4426950408889634)
        d = 1.0 + e
        r = 1.0 / d
        lg = jnp.log2(d) * 0.6931471805599453
        m = jnp.maximum(x, 0.0)
        w = jnp.where(x >= 0.0, 1.0, e * e)
        acc = acc + (m + lg) * (r * r) * w

    @pl.when(s == 0)
    def _init():
        acc_ref[...] = acc

    @pl.when(s > 0)
    def _accum():
        acc_ref[...] += acc

    cx = anc_ref[0]
    cy = anc_ref[1]
    st = anc_ref[2]
    gi = (jax.lax.broadcasted_iota(jnp.int32, (rows, lanes), 0) * lanes
          + jax.lax.broadcasted_iota(jnp.int32, (rows, lanes), 1))

    for j in range(_MB):
        b = s * _MB + j
        label = labels_ref[b]
        gcx = gt_ref[b, 0]
        gcy = gt_ref[b, 1]
        gw = gt_ref[b, 2]
        gh = gt_ref[b, 3]
        gx1 = gcx - gw * 0.5
        gy1 = gcy - gh * 0.5
        gx2 = gcx + gw * 0.5
        gy2 = gcy + gh * 0.5

        d0 = jnp.maximum(boxes_ref[j, 0], 0.0) * st
        d1 = jnp.maximum(boxes_ref[j, 1], 0.0) * st
        d2 = jnp.maximum(boxes_ref[j, 2], 0.0) * st
        d3 = jnp.maximum(boxes_ref[j, 3], 0.0) * st
        px1 = cx - d0
        py1 = cy - d1
        px2 = cx + d2
        py2 = cy + d3

        area1 = (px2 - px1) * (py2 - py1)
        area2 = (gx2 - gx1) * (gy2 - gy1)
        iw = jnp.maximum(jnp.minimum(px2, gx2) - jnp.maximum(px1, gx1), 0.0)
        ih = jnp.maximum(jnp.minimum(py2, gy2) - jnp.maximum(py1, gy1), 0.0)
        inter = iw * ih
        iou = jnp.maximum(inter / (area1 + area2 - inter), 1e-9)

        is_in = ((cx >= gx1) & (cx <= gx2)) & ((cy >= gy1) & (cy <= gy2))
        dist = (cx - (gx1 + gx2) * 0.5) ** 2 + (cy - (gy1 + gy2) * 0.5) ** 2
        dmin = jnp.min(dist)
        fb_idx = jnp.min(jnp.where(dist == dmin, gi, num_a))
        any_in = jnp.any(is_in)
        validm = jnp.logical_or(jnp.logical_and(any_in, is_in),
                                jnp.logical_and(jnp.logical_not(any_in),
                                                gi == fb_idx))

        row = scores_ref[0, pl.ds(j * slab + label * rows, rows)]
        e_row = jnp.exp(-jnp.abs(row))
        r_row = 1.0 / (1.0 + e_row)
        sig = jnp.where(row >= 0.0, r_row, e_row * r_row)
        iou2 = iou * iou
        iou6 = iou2 * iou2 * iou2
        metric = jnp.where(validm, jnp.sqrt(sig) * iou6, _NEG_INF)

        met_s[b] = metric
        iou_s[b] = iou
        row_s[b] = row
        px1_s[b] = px1
        py1_s[b] = py1
        px2_s[b] = px2
        py2_s[b] = py2

    @pl.when(s == ns - 1)
    def _finish():
        bi = jax.lax.broadcasted_iota(jnp.int32, (nb, 1, 1), 0)
        gcxv = jnp.zeros((nb, 1, 1), jnp.float32)
        gcyv = jnp.zeros((nb, 1, 1), jnp.float32)
        gwv = jnp.zeros((nb, 1, 1), jnp.float32)
        ghv = jnp.zeros((nb, 1, 1), jnp.float32)
        for k in range(nb):
            sel = (bi == k).astype(jnp.float32)
            gcxv += sel * gt_ref[k, 0]
            gcyv += sel * gt_ref[k, 1]
            gwv += sel * gt_ref[k, 2]
            ghv += sel * gt_ref[k, 3]
        gx1v = gcxv - gwv * 0.5
        gy1v = gcyv - ghv * 0.5
        gx2v = gcxv + gwv * 0.5
        gy2v = gcyv + ghv * 0.5
        vgt = jnp.where((gx2v > gx1v) & (gy2v > gy1v), 1.0, 0.0)

        met3 = met_s[...]
        gi3 = (jax.lax.broadcasted_iota(jnp.int32, (1, rows, lanes), 1)
               * lanes
               + jax.lax.broadcasted_iota(jnp.int32, (1, rows, lanes), 2))
        gi3 = jnp.broadcast_to(gi3, (nb, rows, lanes))

        def body(_, carry):
            met, fg_m = carry
            m = jnp.max(met, axis=(1, 2), keepdims=True)
            cand = jnp.where(met == m, gi3, num_a)
            sidx = jnp.min(cand, axis=(1, 2), keepdims=True)
            onehot = gi3 == sidx
            takef = jnp.where(m > _NEG_INF, 1.0, 0.0) * vgt
            fg_m = fg_m + onehot.astype(jnp.float32) * takef
            met = jnp.where(onehot, _NEG_INF, met)
            return met, fg_m

        fg0 = jnp.zeros((nb, rows, lanes), jnp.float32)
        _, fgf = jax.lax.fori_loop(0, _TOPK, body, (met3, fg0))

        iou3 = iou_s[...]
        row3 = row_s[...]

        t = iou3 * fgf
        e3 = jnp.exp(-jnp.abs(row3))
        r3 = 1.0 / (1.0 + e3)
        sig3 = jnp.where(row3 >= 0.0, r3, e3 * r3)
        l1p = jnp.log1p(e3)
        mx = jnp.maximum(row3, 0.0)
        ce_t = mx - row3 * t + l1p
        p_t = sig3 * t + (1.0 - sig3) * (1.0 - t)
        f_t = (0.25 * t + 0.75 * (1.0 - t)) * ce_t * (1.0 - p_t) * (1.0 - p_t)
        f_0 = 0.75 * (mx + l1p) * sig3 * sig3
        corr = jnp.sum(fgf * (f_t - f_0))

        eps = 1e-7
        q1 = px1_s[...]
        q2 = py1_s[...]
        q3 = px2_s[...]
        q4 = py2_s[...]
        w1 = q3 - q1
        h1 = q4 - q2
        w2 = gx2v - gx1v
        h2 = gy2v - gy1v
        iwv = jnp.maximum(jnp.minimum(q3, gx2v) - jnp.maximum(q1, gx1v), 0.0)
        ihv = jnp.maximum(jnp.minimum(q4, gy2v) - jnp.maximum(q2, gy1v), 0.0)
        inter3 = iwv * ihv
        union = w1 * h1 + w2 * h2 - inter3 + eps
        iou_c = inter3 / union
        cw = jnp.maximum(q3, gx2v) - jnp.minimum(q1, gx1v)
        ch = jnp.maximum(q4, gy2v) - jnp.minimum(q2, gy1v)
        c2 = cw * cw + ch * ch + eps
        rho2 = ((gx1v + gx2v - q1 - q3) ** 2
                + (gy1v + gy2v - q2 - q4) ** 2) * 0.25
        dang = _atan_nonneg(w2 / (h2 + eps)) - _atan_nonneg(w1 / (h1 + eps))
        v = (4.0 / (math.pi ** 2)) * dang * dang
        alpha_t = v / (v - iou_c + (1.0 + eps))
        ciou = 1.0 - iou_c + rho2 / c2 + alpha_t * v
        box_sum = jnp.sum(ciou * fgf)

        num_pos = jnp.maximum(1.0, jnp.sum(fgf))
        loss_cls = (0.75 * jnp.sum(acc_ref[...]) + corr) / num_pos
        loss_box = box_sum / num_pos
        out_ref[0] = loss_cls + 2.5 * loss_box
        out_ref[1] = loss_cls
        out_ref[2] = loss_box


def _build_anchors(shapes, strides=(8, 16, 32)):
    cxs, cys, sts = [], [], []
    for (h, w), s in zip(shapes, strides):
        gy, gx = np.meshgrid(np.arange(h, dtype=np.float32),
                             np.arange(w, dtype=np.float32), indexing="ij")
        cxs.append((gx.reshape(-1) + 0.5) * s)
        cys.append((gy.reshape(-1) + 0.5) * s)
        sts.append(np.full(h * w, float(s), np.float32))
    return np.stack([np.concatenate(cxs), np.concatenate(cys),
                     np.concatenate(sts)]).astype(np.float32)


def kernel(boxes, scores, feats_p3, feats_p4, feats_p5, gt_bboxes_cxcywh,
           gt_labels):
    bsz, nc, num_a = scores.shape
    rows = num_a // _LANES
    shapes = [feats_p3.shape[2:], feats_p4.shape[2:], feats_p5.shape[2:]]
    anc = jnp.asarray(_build_anchors(shapes).reshape(3, rows, _LANES))

    scores3 = scores.reshape(bsz // _MB, _MB * nc * rows, _LANES)
    boxes4 = boxes.reshape(bsz, 4, rows, _LANES)
    labels = gt_labels.astype(jnp.int32)
    gt = gt_bboxes_cxcywh.astype(jnp.float32)

    plane = pltpu.VMEM((bsz, rows, _LANES), jnp.float32)
    out = pl.pallas_call(
        _loss_kernel,
        grid=(bsz // _MB,),
        in_specs=[
            pl.BlockSpec(memory_space=pltpu.SMEM),
            pl.BlockSpec(memory_space=pltpu.SMEM),
            pl.BlockSpec((1, _MB * nc * rows, _LANES), lambda s: (s, 0, 0)),
            pl.BlockSpec((_MB, 4, rows, _LANES), lambda s: (s, 0, 0, 0)),
            pl.BlockSpec((3, rows, _LANES), lambda s: (0, 0, 0)),
        ],
        out_specs=pl.BlockSpec(memory_space=pltpu.SMEM),
        out_shape=jax.ShapeDtypeStruct((3,), jnp.float32),
        scratch_shapes=[pltpu.VMEM((rows, _LANES), jnp.float32),
                        plane, plane, plane, plane, plane, plane, plane],
        compiler_params=pltpu.CompilerParams(
            dimension_semantics=("arbitrary",)),
    )(labels, gt, scores3, boxes4, anc)

    return out[0], out[1], out[2]


# speedup vs baseline: 1.0219x; 1.0219x over previous
import math

import jax
import jax.numpy as jnp
import numpy as np
from jax.experimental import pallas as pl
from jax.experimental.pallas import tpu as pltpu

_LANES = 128
_TOPK = 10
_MB = 2
_NEG_INF = float("-inf")
_HALF_PI = math.pi / 2.0

_ATAN_COEFS = (1.0, -0.33333322, 0.19999558, -0.14278576, 0.11050771,
               -0.08785043, 0.06685281, -0.04392841, 0.02191294,
               -0.00703067, 0.00105761)


def _atan_nonneg(z):
    inv = z > 1.0
    zz = jnp.where(inv, 1.0 / jnp.maximum(z, 1e-30), z)
    u = zz * zz
    q = jnp.full_like(u, _ATAN_COEFS[-1])
    for c in _ATAN_COEFS[-2::-1]:
        q = q * u + c
    r = zz * q
    return jnp.where(inv, _HALF_PI - r, r)


def _loss_kernel(labels_ref, gt_ref, scores_ref, boxes_ref, anc_ref,
                 out_ref, acc_ref, met_s, iou_s, row_s,
                 px1_s, py1_s, px2_s, py2_s):
    s = pl.program_id(0)
    ns = pl.num_programs(0)
    rows, lanes = anc_ref.shape[1], anc_ref.shape[2]
    num_a = rows * lanes
    nb = ns * _MB
    slab = scores_ref.shape[1] // _MB

    crows = 2 * rows
    acc = jnp.zeros((crows, lanes), jnp.float32)
    for i in range(_MB * slab // crows):
        x = scores_ref[0, i * crows:(i + 1) * crows]
        e = jnp.exp2(jnp.abs(x) * -1.4426950408889634)
        d = 1.0 + e
        r = 1.0 / d
        lg = jnp.log2(d) * 0.6931471805599453
        m = jnp.maximum(x, 0.0)
        w = jnp.where(x >= 0.0, 1.0, e * e)
        acc = acc + (m + lg) * (r * r) * w

    @pl.when(s == 0)
    def _init():
        acc_ref[...] = acc

    @pl.when(s > 0)
    def _accum():
        acc_ref[...] += acc

    cx = anc_ref[0]
    cy = anc_ref[1]
    st = anc_ref[2]
    gi = (jax.lax.broadcasted_iota(jnp.int32, (rows, lanes), 0) * lanes
          + jax.lax.broadcasted_iota(jnp.int32, (rows, lanes), 1))

    for j in range(_MB):
        b = s * _MB + j
        label = labels_ref[b]
        gcx = gt_ref[b, 0]
        gcy = gt_ref[b, 1]
        gw = gt_ref[b, 2]
        gh = gt_ref[b, 3]
        gx1 = gcx - gw * 0.5
        gy1 = gcy - gh * 0.5
        gx2 = gcx + gw * 0.5
        gy2 = gcy + gh * 0.5

        d0 = jnp.maximum(boxes_ref[j, 0], 0.0) * st
        d1 = jnp.maximum(boxes_ref[j, 1], 0.0) * st
        d2 = jnp.maximum(boxes_ref[j, 2], 0.0) * st
        d3 = jnp.maximum(boxes_ref[j, 3], 0.0) * st
        px1 = cx - d0
        py1 = cy - d1
        px2 = cx + d2
        py2 = cy + d3

        area1 = (px2 - px1) * (py2 - py1)
        area2 = (gx2 - gx1) * (gy2 - gy1)
        iw = jnp.maximum(jnp.minimum(px2, gx2) - jnp.maximum(px1, gx1), 0.0)
        ih = jnp.maximum(jnp.minimum(py2, gy2) - jnp.maximum(py1, gy1), 0.0)
        inter = iw * ih
        iou = jnp.maximum(inter / (area1 + area2 - inter), 1e-9)

        is_in = ((cx >= gx1) & (cx <= gx2)) & ((cy >= gy1) & (cy <= gy2))
        dist = (cx - (gx1 + gx2) * 0.5) ** 2 + (cy - (gy1 + gy2) * 0.5) ** 2
        dmin = jnp.min(dist)
        fb_idx = jnp.min(jnp.where(dist == dmin, gi, num_a))
        any_in = jnp.any(is_in)
        validm = jnp.logical_or(jnp.logical_and(any_in, is_in),
                                jnp.logical_and(jnp.logical_not(any_in),
                                                gi == fb_idx))

        row = scores_ref[0, pl.ds(j * slab + label * rows, rows)]
        e_row = jnp.exp(-jnp.abs(row))
        r_row = 1.0 / (1.0 + e_row)
        sig = jnp.where(row >= 0.0, r_row, e_row * r_row)
        iou2 = iou * iou
        iou6 = iou2 * iou2 * iou2
        metric = jnp.where(validm, jnp.sqrt(sig) * iou6, _NEG_INF)

        met_s[b] = metric
        iou_s[b] = iou
        row_s[b] = row
        px1_s[b] = px1
        py1_s[b] = py1
        px2_s[b] = px2
        py2_s[b] = py2

    @pl.when(s == ns - 1)
    def _finish():
        bi = jax.lax.broadcasted_iota(jnp.int32, (nb, 1, 1), 0)
        gcxv = jnp.zeros((nb, 1, 1), jnp.float32)
        gcyv = jnp.zeros((nb, 1, 1), jnp.float32)
        gwv = jnp.zeros((nb, 1, 1), jnp.float32)
        ghv = jnp.zeros((nb, 1, 1), jnp.float32)
        for k in range(nb):
            sel = (bi == k).astype(jnp.float32)
            gcxv += sel * gt_ref[k, 0]
            gcyv += sel * gt_ref[k, 1]
            gwv += sel * gt_ref[k, 2]
            ghv += sel * gt_ref[k, 3]
        gx1v = gcxv - gwv * 0.5
        gy1v = gcyv - ghv * 0.5
        gx2v = gcxv + gwv * 0.5
        gy2v = gcyv + ghv * 0.5
        vgt = jnp.where((gx2v > gx1v) & (gy2v > gy1v), 1.0, 0.0)

        met3 = met_s[...]
        gi3 = (jax.lax.broadcasted_iota(jnp.int32, (1, rows, lanes), 1)
               * lanes
               + jax.lax.broadcasted_iota(jnp.int32, (1, rows, lanes), 2))
        gi3 = jnp.broadcast_to(gi3, (nb, rows, lanes))

        def body(_, carry):
            met, fg_m = carry
            m = jnp.max(met, axis=(1, 2), keepdims=True)
            cand = jnp.where(met == m, gi3, num_a)
            sidx = jnp.min(cand, axis=(1, 2), keepdims=True)
            onehot = gi3 == sidx
            takef = jnp.where(m > _NEG_INF, 1.0, 0.0) * vgt
            fg_m = fg_m + onehot.astype(jnp.float32) * takef
            met = jnp.where(onehot, _NEG_INF, met)
            return met, fg_m

        fg0 = jnp.zeros((nb, rows, lanes), jnp.float32)
        _, fgf = jax.lax.fori_loop(0, _TOPK, body, (met3, fg0))

        iou3 = iou_s[...]
        row3 = row_s[...]

        t = iou3 * fgf
        e3 = jnp.exp(-jnp.abs(row3))
        r3 = 1.0 / (1.0 + e3)
        sig3 = jnp.where(row3 >= 0.0, r3, e3 * r3)
        l1p = jnp.log1p(e3)
        mx = jnp.maximum(row3, 0.0)
        ce_t = mx - row3 * t + l1p
        p_t = sig3 * t + (1.0 - sig3) * (1.0 - t)
        f_t = (0.25 * t + 0.75 * (1.0 - t)) * ce_t * (1.0 - p_t) * (1.0 - p_t)
        f_0 = 0.75 * (mx + l1p) * sig3 * sig3
        corr = jnp.sum(fgf * (f_t - f_0))

        eps = 1e-7
        q1 = px1_s[...]
        q2 = py1_s[...]
        q3 = px2_s[...]
        q4 = py2_s[...]
        w1 = q3 - q1
        h1 = q4 - q2
        w2 = gx2v - gx1v
        h2 = gy2v - gy1v
        iwv = jnp.maximum(jnp.minimum(q3, gx2v) - jnp.maximum(q1, gx1v), 0.0)
        ihv = jnp.maximum(jnp.minimum(q4, gy2v) - jnp.maximum(q2, gy1v), 0.0)
        inter3 = iwv * ihv
        union = w1 * h1 + w2 * h2 - inter3 + eps
        iou_c = inter3 / union
        cw = jnp.maximum(q3, gx2v) - jnp.minimum(q1, gx1v)
        ch = jnp.maximum(q4, gy2v) - jnp.minimum(q2, gy1v)
        c2 = cw * cw + ch * ch + eps
        rho2 = ((gx1v + gx2v - q1 - q3) ** 2
                + (gy1v + gy2v - q2 - q4) ** 2) * 0.25
        dang = _atan_nonneg(w2 / (h2 + eps)) - _atan_nonneg(w1 / (h1 + eps))
        v = (4.0 / (math.pi ** 2)) * dang * dang
        alpha_t = v / (v - iou_c + (1.0 + eps))
        ciou = 1.0 - iou_c + rho2 / c2 + alpha_t * v
        box_sum = jnp.sum(ciou * fgf)

        num_pos = jnp.maximum(1.0, jnp.sum(fgf))
        loss_cls = (0.75 * jnp.sum(acc_ref[...]) + corr) / num_pos
        loss_box = box_sum / num_pos
        out_ref[0] = loss_cls + 2.5 * loss_box
        out_ref[1] = loss_cls
        out_ref[2] = loss_box


def _build_anchors(shapes, strides=(8, 16, 32)):
    cxs, cys, sts = [], [], []
    for (h, w), s in zip(shapes, strides):
        gy, gx = np.meshgrid(np.arange(h, dtype=np.float32),
                             np.arange(w, dtype=np.float32), indexing="ij")
        cxs.append((gx.reshape(-1) + 0.5) * s)
        cys.append((gy.reshape(-1) + 0.5) * s)
        sts.append(np.full(h * w, float(s), np.float32))
    return np.stack([np.concatenate(cxs), np.concatenate(cys),
                     np.concatenate(sts)]).astype(np.float32)


def kernel(boxes, scores, feats_p3, feats_p4, feats_p5, gt_bboxes_cxcywh,
           gt_labels):
    bsz, nc, num_a = scores.shape
    rows = num_a // _LANES
    shapes = [feats_p3.shape[2:], feats_p4.shape[2:], feats_p5.shape[2:]]
    anc = jnp.asarray(_build_anchors(shapes).reshape(3, rows, _LANES))

    scores3 = scores.reshape(bsz // _MB, _MB * nc * rows, _LANES)
    boxes4 = boxes.reshape(bsz, 4, rows, _LANES)
    labels = gt_labels.astype(jnp.int32)
    gt = gt_bboxes_cxcywh.astype(jnp.float32)

    plane = pltpu.VMEM((bsz, rows, _LANES), jnp.float32)
    out = pl.pallas_call(
        _loss_kernel,
        grid=(bsz // _MB,),
        in_specs=[
            pl.BlockSpec(memory_space=pltpu.SMEM),
            pl.BlockSpec(memory_space=pltpu.SMEM),
            pl.BlockSpec((1, _MB * nc * rows, _LANES), lambda s: (s, 0, 0)),
            pl.BlockSpec((_MB, 4, rows, _LANES), lambda s: (s, 0, 0, 0)),
            pl.BlockSpec((3, rows, _LANES), lambda s: (0, 0, 0)),
        ],
        out_specs=pl.BlockSpec(memory_space=pltpu.SMEM),
        out_shape=jax.ShapeDtypeStruct((3,), jnp.float32),
        scratch_shapes=[pltpu.VMEM((2 * rows, _LANES), jnp.float32),
                        plane, plane, plane, plane, plane, plane, plane],
        compiler_params=pltpu.CompilerParams(
            dimension_semantics=("arbitrary",)),
    )(labels, gt, scores3, boxes4, anc)

    return out[0], out[1], out[2]
